# Initial kernel scaffold; baseline (speedup 1.0000x reference)
import jax
import jax.numpy as jnp
from jax.experimental import pallas as pl


def kernel(edge_costs, t12, t13, t23, corr_12, corr_13, corr_23, edge_counter, edge_index, W1, b1, W2, b2, Wout, bout):
    raise NotImplementedError("write your pallas kernel here")


# speedup vs baseline: 30.4372x; 30.4372x over previous
import functools

import jax
import jax.numpy as jnp
from jax import lax
from jax.experimental import pallas as pl
from jax.experimental.pallas import tpu as pltpu
from jax.experimental.pallas import tpu_sc as plsc

T = 50000
M = 800000
E = 800000
HID = 32
NC = 2
NS = 16
NW = NC * NS
WIN = 128
TP = T + WIN

K = 4
SBE = K * WIN
WIN2 = 512
EPW = 25088
EP = NW * EPW
NSB = 49

CPW = 4736
CP = NW * CPW
NSBC = 9
TAILC = 1

_mesh = plsc.VectorSubcoreMesh(core_axis_name="c", subcore_axis_name="s",
                               num_cores=NC, num_subcores=NS)


@functools.partial(
    pl.kernel,
    out_type=(
        jax.ShapeDtypeStruct((CP,), jnp.float32),
        jax.ShapeDtypeStruct((NC, T), jnp.float32),
    ),
    mesh=_mesh,
    scratch_types=(
        pltpu.VMEM((K, WIN), jnp.int32),
        pltpu.VMEM((K, WIN), jnp.float32),
        pltpu.VMEM((K, WIN), jnp.float32),
        pltpu.VMEM((K, WIN), jnp.float32),
        pltpu.VMEM((K, WIN), jnp.int32),
        pltpu.VMEM((WIN,), jnp.float32),
        pltpu.VMEM_SHARED((TP,), jnp.float32),
        pltpu.SemaphoreType.DMA,
        pltpu.SemaphoreType.DMA,
    ),
    compiler_params=pltpu.CompilerParams(use_tc_tiling_on_sc=False),
)
def _sc_phase1(ec_hbm, cnt_hbm, corr_hbm, t_hbm, dst_hbm, ztp_hbm,
               tout_hbm, deg_hbm,
               corr_v, ecg_v, cntg_v, t_v, dst_v, ones_v, deg_sh, gsem, ssem):
    c = lax.axis_index("c")
    s = lax.axis_index("s")
    wid = s * NC + c

    @pl.when(s == 0)
    def _():
        pltpu.sync_copy(ztp_hbm, deg_sh)

    for i in range(WIN // 16):
        ones_v[pl.ds(i * 16, 16)] = jnp.full((16,), 1.0, jnp.float32)

    plsc.subcore_barrier()

    def corr_superbatch(base, nk):
        descs = []
        for kk in range(nk):
            descs.append(pltpu.async_copy(
                corr_hbm.at[pl.ds(base + kk * WIN, WIN)], corr_v.at[kk], gsem))
            descs.append(pltpu.async_copy(
                t_hbm.at[pl.ds(base + kk * WIN, WIN)], t_v.at[kk], gsem))
        for d in descs:
            d.wait()
        descs = []
        for kk in range(nk):
            descs.append(pltpu.async_copy(
                ec_hbm.at[corr_v.at[kk]], ecg_v.at[kk], gsem))
            descs.append(pltpu.async_copy(
                cnt_hbm.at[corr_v.at[kk]], cntg_v.at[kk], gsem))
        for d in descs:
            d.wait()
        for kk in range(nk):
            for i in range(WIN // 16):
                ds = pl.ds(i * 16, 16)
                t_v[kk, ds] = t_v[kk, ds] + ecg_v[kk, ds] / cntg_v[kk, ds]
        descs = []
        for kk in range(nk):
            descs.append(pltpu.async_copy(
                t_v.at[kk], tout_hbm.at[pl.ds(base + kk * WIN, WIN)], ssem))
        for d in descs:
            d.wait()

    cbase = wid * CPW

    def corr_body(j, _):
        corr_superbatch(cbase + j * K * WIN, K)
        return 0

    lax.fori_loop(0, NSBC, corr_body, 0)
    corr_superbatch(cbase + NSBC * K * WIN, TAILC)

    ebase = wid * EPW

    def deg_superbatch(base, nk):
        descs = []
        for kk in range(nk):
            descs.append(pltpu.async_copy(
                dst_hbm.at[pl.ds(base + kk * WIN, WIN)], dst_v.at[kk], gsem))
        for d in descs:
            d.wait()
        descs = []
        for kk in range(nk):
            descs.append(pltpu.async_copy(
                ones_v, deg_sh.at[dst_v.at[kk]], ssem, add=True))
        for d in descs:
            d.wait()

    def deg_body(j, _):
        deg_superbatch(ebase + j * SBE, K)
        return 0

    lax.fori_loop(0, NSB, deg_body, 0)

    plsc.subcore_barrier()

    @pl.when(s == 0)
    def _():
        pltpu.sync_copy(deg_sh.at[pl.ds(0, T)], deg_hbm.at[c])


@functools.partial(
    pl.kernel,
    out_type=jax.ShapeDtypeStruct((NC, T, HID), jnp.float32),
    mesh=_mesh,
    scratch_types=(
        pltpu.VMEM((1, WIN2), jnp.int32),
        pltpu.VMEM((1, WIN2), jnp.int32),
        pltpu.VMEM((WIN2, HID), jnp.float32),
        pltpu.VMEM_SHARED((TP, HID), jnp.float32),
        pltpu.SemaphoreType.DMA,
        pltpu.SemaphoreType.DMA,
    ),
    compiler_params=pltpu.CompilerParams(use_tc_tiling_on_sc=False),
)
def _sc_agg(y_hbm, src_hbm, dst_hbm, zt_hbm, z_hbm,
            src_v, dst_v, rows_v, z_sh, gsem, ssem):
    c = lax.axis_index("c")
    s = lax.axis_index("s")
    wid = s * NC + c
    ebase = wid * EPW

    @pl.when(s == 0)
    def _():
        pltpu.sync_copy(zt_hbm, z_sh)

    plsc.subcore_barrier()

    def superbatch(base):
        d0 = pltpu.async_copy(src_hbm.at[pl.ds(base, WIN2)], src_v.at[0], gsem)
        d1 = pltpu.async_copy(dst_hbm.at[pl.ds(base, WIN2)], dst_v.at[0], gsem)
        d0.wait()
        d1.wait()
        pltpu.async_copy(y_hbm.at[src_v.at[0]], rows_v, gsem).wait()
        pltpu.async_copy(rows_v, z_sh.at[dst_v.at[0]], ssem, add=True).wait()

    def body(j, _):
        superbatch(ebase + j * WIN2)
        return 0

    lax.fori_loop(0, NSB, body, 0)

    plsc.subcore_barrier()

    @pl.when(s == 0)
    def _():
        pltpu.sync_copy(z_sh.at[pl.ds(0, T)], z_hbm.at[c])


@functools.partial(
    pl.kernel,
    out_type=jax.ShapeDtypeStruct((NC, M), jnp.float32),
    mesh=_mesh,
    scratch_types=(
        pltpu.VMEM((K, WIN), jnp.int32),
        pltpu.VMEM((K, WIN), jnp.float32),
        pltpu.VMEM_SHARED((M,), jnp.float32),
        pltpu.SemaphoreType.DMA,
        pltpu.SemaphoreType.DMA,
    ),
    compiler_params=pltpu.CompilerParams(use_tc_tiling_on_sc=False),
)
def _sc_scatter_upd(corr_hbm, d_hbm, zm_hbm, upd_hbm,
                    corr_v, d_v, upd_sh, gsem, ssem):
    c = lax.axis_index("c")
    s = lax.axis_index("s")
    wid = s * NC + c
    cbase = wid * CPW

    @pl.when(s == 0)
    def _():
        pltpu.sync_copy(zm_hbm, upd_sh)

    plsc.subcore_barrier()

    def superbatch(base, nk):
        descs = []
        for kk in range(nk):
            descs.append(pltpu.async_copy(
                corr_hbm.at[pl.ds(base + kk * WIN, WIN)], corr_v.at[kk], gsem))
            descs.append(pltpu.async_copy(
                d_hbm.at[pl.ds(base + kk * WIN, WIN)], d_v.at[kk], gsem))
        for d in descs:
            d.wait()
        descs = []
        for kk in range(nk):
            descs.append(pltpu.async_copy(
                d_v.at[kk], upd_sh.at[corr_v.at[kk]], ssem, add=True))
        for d in descs:
            d.wait()

    def body(j, _):
        superbatch(cbase + j * K * WIN, K)
        return 0

    lax.fori_loop(0, NSBC, body, 0)
    superbatch(cbase + NSBC * K * WIN, TAILC)

    plsc.subcore_barrier()

    @pl.when(s == 0)
    def _():
        pltpu.sync_copy(upd_sh, upd_hbm.at[c])


def _tc1_body(t3_ref, d0_ref, d1_ref, w1_ref, y1_ref, dinv_ref):
    deg = d0_ref[...] + d1_ref[...] + 1.0
    dinv = lax.rsqrt(deg)
    dinv_ref[...] = dinv
    y = lax.dot_general(t3_ref[...], w1_ref[...],
                        (((0,), (0,)), ((), ())),
                        preferred_element_type=jnp.float32)
    y1_ref[...] = dinv * y


def _tc2_body(z0_ref, z1_ref, y_ref, dinv_ref, b_ref, w2_ref, y2_ref):
    dinv = dinv_ref[...]
    x = jnp.maximum(dinv * (z0_ref[...] + z1_ref[...] + y_ref[...])
                    + b_ref[...], 0.0)
    h = jnp.dot(x, w2_ref[...], preferred_element_type=jnp.float32)
    y2_ref[...] = dinv * h


def _tc3_body(z0_ref, z1_ref, y_ref, dinv_ref, b_ref, wout_ref, boutt_ref,
              t3_ref, deltat_ref, tout3_ref):
    dinv = dinv_ref[...]
    x = jnp.maximum(dinv * (z0_ref[...] + z1_ref[...] + y_ref[...])
                    + b_ref[...], 0.0)
    deltat = lax.dot_general(wout_ref[...], x,
                             (((0,), (1,)), ((), ())),
                             preferred_element_type=jnp.float32)
    deltat = deltat + boutt_ref[...]
    deltat_ref[...] = deltat
    tout3_ref[...] = t3_ref[...] - deltat


def _tc4_body(ec_ref, cnt_ref, u0_ref, u1_ref, out_ref):
    ec = ec_ref[...]
    out_ref[...] = jnp.where(cnt_ref[...] > 0, 0.0, ec) + u0_ref[...] + u1_ref[...]


BLK = 2048
NBLK = (T + BLK - 1) // BLK


def _row_spec(width):
    return pl.BlockSpec((BLK, width), lambda i: (i, 0))


def _full_spec(shape):
    return pl.BlockSpec(shape, lambda i: (0, 0))


def _tc_call(body, in_specs, out_specs, out_shapes, *args):
    return pl.pallas_call(
        body,
        grid=(NBLK,),
        in_specs=in_specs,
        out_specs=out_specs,
        out_shape=out_shapes,
    )(*args)


def kernel(edge_costs, t12, t13, t23, corr_12, corr_13, corr_23, edge_counter, edge_index, W1, b1, W2, b2, Wout, bout):
    f32 = jnp.float32
    idt = edge_index.dtype
    npad = EP - E
    src = edge_index[0]
    dst = edge_index[1]
    src_p = jnp.concatenate([src, jnp.arange(npad, dtype=idt) % T])
    dst_p = jnp.concatenate([dst, T + jnp.arange(npad, dtype=idt) % WIN])

    cpad = CP - 3 * T
    corr_all = jnp.concatenate([
        corr_12, corr_13, corr_23,
        jnp.arange(cpad, dtype=corr_12.dtype) % M,
    ])
    t_all = jnp.concatenate([t12, t13, t23, jnp.zeros((cpad,), f32)])
    zeros_tp = jnp.zeros((TP,), f32)
    zeros_tp32 = jnp.zeros((TP, HID), f32)
    zeros_m = jnp.zeros((M,), f32)

    t_all_new, deg_part = _sc_phase1(
        edge_costs, edge_counter, corr_all, t_all, dst_p, zeros_tp)

    t3 = t_all_new[:3 * T].reshape(3, T)
    d0 = deg_part[0].reshape(T, 1)
    d1 = deg_part[1].reshape(T, 1)

    col3_spec = pl.BlockSpec((3, BLK), lambda i: (0, i))
    y1, dinv = _tc_call(
        _tc1_body,
        [col3_spec, _row_spec(1), _row_spec(1), _full_spec((3, HID))],
        (_row_spec(HID), _row_spec(1)),
        (jax.ShapeDtypeStruct((T, HID), f32), jax.ShapeDtypeStruct((T, 1), f32)),
        t3, d0, d1, W1)

    z1p = _sc_agg(y1, src_p, dst_p, zeros_tp32)

    y2 = _tc_call(
        _tc2_body,
        [_row_spec(HID)] * 3 + [_row_spec(1), _full_spec((1, HID)),
                                _full_spec((HID, HID))],
        _row_spec(HID),
        jax.ShapeDtypeStruct((T, HID), f32),
        z1p[0], z1p[1], y1, dinv, b1.reshape(1, HID), W2)

    z2p = _sc_agg(y2, src_p, dst_p, zeros_tp32)

    deltat, tout3 = _tc_call(
        _tc3_body,
        [_row_spec(HID)] * 3 + [_row_spec(1), _full_spec((1, HID)),
                                _full_spec((HID, 3)), _full_spec((3, 1)),
                                col3_spec],
        (col3_spec, col3_spec),
        (jax.ShapeDtypeStruct((3, T), f32), jax.ShapeDtypeStruct((3, T), f32)),
        z2p[0], z2p[1], y2, dinv, b2.reshape(1, HID), Wout,
        bout.reshape(3, 1), t3)

    t12o = tout3[0]
    t13o = tout3[1]
    t23o = tout3[2]

    d_all = jnp.concatenate([deltat.reshape(3 * T), jnp.zeros((cpad,), f32)])

    updp = _sc_scatter_upd(corr_all, d_all, zeros_m)

    SH = 6250
    ec_out = pl.pallas_call(
        _tc4_body,
        out_shape=jax.ShapeDtypeStruct((SH, WIN), f32),
    )(edge_costs.reshape(SH, WIN), edge_counter.reshape(SH, WIN),
      updp[0].reshape(SH, WIN), updp[1].reshape(SH, WIN))

    return (ec_out.reshape(M), t12o, t13o, t23o)


# speedup vs baseline: 34.0616x; 1.1191x over previous
import functools

import jax
import jax.numpy as jnp
from jax import lax
from jax.experimental import pallas as pl
from jax.experimental.pallas import tpu as pltpu
from jax.experimental.pallas import tpu_sc as plsc

T = 50000
M = 800000
E = 800000
HID = 32
NC = 2
NS = 16
NW = NC * NS
WIN = 128
TP = T + WIN

K = 4
SBE = K * WIN
WIN2 = 512
WINA = 256
NSET = 3
EPW = 25088
EP = NW * EPW
NSB = 49
NSTEP = EPW // WINA

CPW = 4736
CP = NW * CPW
NSBC = 9
TAILC = 1

_mesh = plsc.VectorSubcoreMesh(core_axis_name="c", subcore_axis_name="s",
                               num_cores=NC, num_subcores=NS)


@functools.partial(
    pl.kernel,
    out_type=(
        jax.ShapeDtypeStruct((CP,), jnp.float32),
        jax.ShapeDtypeStruct((NC, T), jnp.float32),
    ),
    mesh=_mesh,
    scratch_types=(
        pltpu.VMEM((K, WIN), jnp.int32),
        pltpu.VMEM((K, WIN), jnp.float32),
        pltpu.VMEM((K, WIN), jnp.float32),
        pltpu.VMEM((K, WIN), jnp.float32),
        pltpu.VMEM((K, WIN), jnp.int32),
        pltpu.VMEM((WIN,), jnp.float32),
        pltpu.VMEM_SHARED((TP,), jnp.float32),
        pltpu.SemaphoreType.DMA,
        pltpu.SemaphoreType.DMA,
    ),
    compiler_params=pltpu.CompilerParams(use_tc_tiling_on_sc=False),
)
def _sc_phase1(ec_hbm, cnt_hbm, corr_hbm, t_hbm, dst_hbm, ztp_hbm,
               tout_hbm, deg_hbm,
               corr_v, ecg_v, cntg_v, t_v, dst_v, ones_v, deg_sh, gsem, ssem):
    c = lax.axis_index("c")
    s = lax.axis_index("s")
    wid = s * NC + c

    @pl.when(s == 0)
    def _():
        pltpu.sync_copy(ztp_hbm, deg_sh)

    for i in range(WIN // 16):
        ones_v[pl.ds(i * 16, 16)] = jnp.full((16,), 1.0, jnp.float32)

    plsc.subcore_barrier()

    def corr_superbatch(base, nk):
        descs = []
        for kk in range(nk):
            descs.append(pltpu.async_copy(
                corr_hbm.at[pl.ds(base + kk * WIN, WIN)], corr_v.at[kk], gsem))
            descs.append(pltpu.async_copy(
                t_hbm.at[pl.ds(base + kk * WIN, WIN)], t_v.at[kk], gsem))
        for d in descs:
            d.wait()
        descs = []
        for kk in range(nk):
            descs.append(pltpu.async_copy(
                ec_hbm.at[corr_v.at[kk]], ecg_v.at[kk], gsem))
            descs.append(pltpu.async_copy(
                cnt_hbm.at[corr_v.at[kk]], cntg_v.at[kk], gsem))
        for d in descs:
            d.wait()
        for kk in range(nk):
            for i in range(WIN // 16):
                ds = pl.ds(i * 16, 16)
                t_v[kk, ds] = t_v[kk, ds] + ecg_v[kk, ds] / cntg_v[kk, ds]
        descs = []
        for kk in range(nk):
            descs.append(pltpu.async_copy(
                t_v.at[kk], tout_hbm.at[pl.ds(base + kk * WIN, WIN)], ssem))
        for d in descs:
            d.wait()

    cbase = wid * CPW

    def corr_body(j, _):
        corr_superbatch(cbase + j * K * WIN, K)
        return 0

    lax.fori_loop(0, NSBC, corr_body, 0)
    corr_superbatch(cbase + NSBC * K * WIN, TAILC)

    ebase = wid * EPW

    def deg_superbatch(base, nk):
        descs = []
        for kk in range(nk):
            descs.append(pltpu.async_copy(
                dst_hbm.at[pl.ds(base + kk * WIN, WIN)], dst_v.at[kk], gsem))
        for d in descs:
            d.wait()
        descs = []
        for kk in range(nk):
            descs.append(pltpu.async_copy(
                ones_v, deg_sh.at[dst_v.at[kk]], ssem, add=True))
        for d in descs:
            d.wait()

    def deg_body(j, _):
        deg_superbatch(ebase + j * SBE, K)
        return 0

    lax.fori_loop(0, NSB, deg_body, 0)

    plsc.subcore_barrier()

    @pl.when(s == 0)
    def _():
        pltpu.sync_copy(deg_sh.at[pl.ds(0, T)], deg_hbm.at[c])


@functools.partial(
    pl.kernel,
    out_type=jax.ShapeDtypeStruct((NC, T, HID), jnp.float32),
    mesh=_mesh,
    scratch_types=(
        pltpu.VMEM((NSET, WINA), jnp.int32),
        pltpu.VMEM((NSET, WINA), jnp.int32),
        pltpu.VMEM((NSET, WINA, HID), jnp.float32),
        pltpu.VMEM_SHARED((TP, HID), jnp.float32),
        pltpu.SemaphoreType.DMA,
        pltpu.SemaphoreType.DMA,
        pltpu.SemaphoreType.DMA,
        pltpu.SemaphoreType.DMA,
        pltpu.SemaphoreType.DMA,
        pltpu.SemaphoreType.DMA,
    ),
    compiler_params=pltpu.CompilerParams(use_tc_tiling_on_sc=False),
)
def _sc_agg(y_hbm, src_hbm, dst_hbm, zt_hbm, z_hbm,
            src_v, dst_v, rows_v, z_sh,
            gsem_0, gsem_1, gsem_2, ssem_0, ssem_1, ssem_2):
    c = lax.axis_index("c")
    s = lax.axis_index("s")
    wid = s * NC + c
    ebase = wid * EPW
    gsem = (gsem_0, gsem_1, gsem_2)
    ssem = (ssem_0, ssem_1, ssem_2)

    @pl.when(s == 0)
    def _():
        pltpu.sync_copy(zt_hbm, z_sh)

    plsc.subcore_barrier()

    def fire_gathers(step, S):
        base = ebase + step * WINA
        d0 = pltpu.async_copy(src_hbm.at[pl.ds(base, WINA)], src_v.at[S],
                              gsem[S])
        d1 = pltpu.async_copy(dst_hbm.at[pl.ds(base, WINA)], dst_v.at[S],
                              gsem[S])
        d0.wait()
        d1.wait()
        pltpu.async_copy(y_hbm.at[src_v.at[S]], rows_v.at[S], gsem[S])

    def drain_gathers(S):
        pltpu.make_async_copy(y_hbm.at[src_v.at[S]], rows_v.at[S],
                              gsem[S]).wait()

    def fire_scatters(S):
        pltpu.async_copy(rows_v.at[S], z_sh.at[dst_v.at[S]], ssem[S],
                         add=True)

    def drain_scatters(S):
        pltpu.make_async_copy(rows_v.at[S], z_sh.at[dst_v.at[S]],
                              ssem[S]).wait()

    fire_gathers(0, 0)
    fire_gathers(1, 1)
    drain_gathers(0)
    fire_scatters(0)
    fire_gathers(2, 2)
    drain_gathers(1)
    fire_scatters(1)

    def body(j, _):
        x0 = 3 + 3 * j
        for dx in range(3):
            S = dx
            drain_scatters(S)
            fire_gathers(x0 + dx, S)
            drain_gathers((S + 2) % 3)
            fire_scatters((S + 2) % 3)
        return 0

    lax.fori_loop(0, (NSTEP - 5) // 3, body, 0)

    drain_scatters(0)
    fire_gathers(NSTEP - 2, 0)
    drain_gathers(2)
    fire_scatters(2)
    drain_scatters(1)
    fire_gathers(NSTEP - 1, 1)
    drain_gathers(0)
    fire_scatters(0)
    drain_gathers(1)
    fire_scatters(1)
    drain_scatters(2)
    drain_scatters(0)
    drain_scatters(1)

    plsc.subcore_barrier()

    @pl.when(s == 0)
    def _():
        pltpu.sync_copy(z_sh.at[pl.ds(0, T)], z_hbm.at[c])


@functools.partial(
    pl.kernel,
    out_type=jax.ShapeDtypeStruct((NC, M), jnp.float32),
    mesh=_mesh,
    scratch_types=(
        pltpu.VMEM((K, WIN), jnp.int32),
        pltpu.VMEM((K, WIN), jnp.float32),
        pltpu.VMEM_SHARED((M,), jnp.float32),
        pltpu.SemaphoreType.DMA,
        pltpu.SemaphoreType.DMA,
    ),
    compiler_params=pltpu.CompilerParams(use_tc_tiling_on_sc=False),
)
def _sc_scatter_upd(corr_hbm, d_hbm, zm_hbm, upd_hbm,
                    corr_v, d_v, upd_sh, gsem, ssem):
    c = lax.axis_index("c")
    s = lax.axis_index("s")
    wid = s * NC + c
    cbase = wid * CPW

    @pl.when(s == 0)
    def _():
        pltpu.sync_copy(zm_hbm, upd_sh)

    plsc.subcore_barrier()

    def superbatch(base, nk):
        descs = []
        for kk in range(nk):
            descs.append(pltpu.async_copy(
                corr_hbm.at[pl.ds(base + kk * WIN, WIN)], corr_v.at[kk], gsem))
            descs.append(pltpu.async_copy(
                d_hbm.at[pl.ds(base + kk * WIN, WIN)], d_v.at[kk], gsem))
        for d in descs:
            d.wait()
        descs = []
        for kk in range(nk):
            descs.append(pltpu.async_copy(
                d_v.at[kk], upd_sh.at[corr_v.at[kk]], ssem, add=True))
        for d in descs:
            d.wait()

    def body(j, _):
        superbatch(cbase + j * K * WIN, K)
        return 0

    lax.fori_loop(0, NSBC, body, 0)
    superbatch(cbase + NSBC * K * WIN, TAILC)

    plsc.subcore_barrier()

    @pl.when(s == 0)
    def _():
        pltpu.sync_copy(upd_sh, upd_hbm.at[c])


def _tc1_body(t3_ref, d0_ref, d1_ref, w1_ref, y1_ref, dinv_ref):
    deg = d0_ref[...] + d1_ref[...] + 1.0
    dinv = lax.rsqrt(deg)
    dinv_ref[...] = dinv
    y = lax.dot_general(t3_ref[...], w1_ref[...],
                        (((0,), (0,)), ((), ())),
                        preferred_element_type=jnp.float32)
    y1_ref[...] = dinv * y


def _tc2_body(z0_ref, z1_ref, y_ref, dinv_ref, b_ref, w2_ref, y2_ref):
    dinv = dinv_ref[...]
    x = jnp.maximum(dinv * (z0_ref[...] + z1_ref[...] + y_ref[...])
                    + b_ref[...], 0.0)
    h = jnp.dot(x, w2_ref[...], preferred_element_type=jnp.float32)
    y2_ref[...] = dinv * h


def _tc3_body(z0_ref, z1_ref, y_ref, dinv_ref, b_ref, wout_ref, boutt_ref,
              t3_ref, deltat_ref, tout3_ref):
    dinv = dinv_ref[...]
    x = jnp.maximum(dinv * (z0_ref[...] + z1_ref[...] + y_ref[...])
                    + b_ref[...], 0.0)
    deltat = lax.dot_general(wout_ref[...], x,
                             (((0,), (1,)), ((), ())),
                             preferred_element_type=jnp.float32)
    deltat = deltat + boutt_ref[...]
    deltat_ref[...] = deltat
    tout3_ref[...] = t3_ref[...] - deltat


def _tc4_body(ec_ref, cnt_ref, u0_ref, u1_ref, out_ref):
    ec = ec_ref[...]
    out_ref[...] = jnp.where(cnt_ref[...] > 0, 0.0, ec) + u0_ref[...] + u1_ref[...]


BLK = 2048
NBLK = (T + BLK - 1) // BLK


def _row_spec(width):
    return pl.BlockSpec((BLK, width), lambda i: (i, 0))


def _full_spec(shape):
    return pl.BlockSpec(shape, lambda i: (0, 0))


def _tc_call(body, in_specs, out_specs, out_shapes, *args):
    return pl.pallas_call(
        body,
        grid=(NBLK,),
        in_specs=in_specs,
        out_specs=out_specs,
        out_shape=out_shapes,
    )(*args)


def kernel(edge_costs, t12, t13, t23, corr_12, corr_13, corr_23, edge_counter, edge_index, W1, b1, W2, b2, Wout, bout):
    f32 = jnp.float32
    idt = edge_index.dtype
    npad = EP - E
    src = edge_index[0]
    dst = edge_index[1]
    src_p = jnp.concatenate([src, jnp.arange(npad, dtype=idt) % T])
    dst_p = jnp.concatenate([dst, T + jnp.arange(npad, dtype=idt) % WIN])

    cpad = CP - 3 * T
    corr_all = jnp.concatenate([
        corr_12, corr_13, corr_23,
        jnp.arange(cpad, dtype=corr_12.dtype) % M,
    ])
    t_all = jnp.concatenate([t12, t13, t23, jnp.zeros((cpad,), f32)])
    zeros_tp = jnp.zeros((TP,), f32)
    zeros_tp32 = jnp.zeros((TP, HID), f32)
    zeros_m = jnp.zeros((M,), f32)

    t_all_new, deg_part = _sc_phase1(
        edge_costs, edge_counter, corr_all, t_all, dst_p, zeros_tp)

    t3 = t_all_new[:3 * T].reshape(3, T)
    d0 = deg_part[0].reshape(T, 1)
    d1 = deg_part[1].reshape(T, 1)

    col3_spec = pl.BlockSpec((3, BLK), lambda i: (0, i))
    y1, dinv = _tc_call(
        _tc1_body,
        [col3_spec, _row_spec(1), _row_spec(1), _full_spec((3, HID))],
        (_row_spec(HID), _row_spec(1)),
        (jax.ShapeDtypeStruct((T, HID), f32), jax.ShapeDtypeStruct((T, 1), f32)),
        t3, d0, d1, W1)

    z1p = _sc_agg(y1, src_p, dst_p, zeros_tp32)

    y2 = _tc_call(
        _tc2_body,
        [_row_spec(HID)] * 3 + [_row_spec(1), _full_spec((1, HID)),
                                _full_spec((HID, HID))],
        _row_spec(HID),
        jax.ShapeDtypeStruct((T, HID), f32),
        z1p[0], z1p[1], y1, dinv, b1.reshape(1, HID), W2)

    z2p = _sc_agg(y2, src_p, dst_p, zeros_tp32)

    deltat, tout3 = _tc_call(
        _tc3_body,
        [_row_spec(HID)] * 3 + [_row_spec(1), _full_spec((1, HID)),
                                _full_spec((HID, 3)), _full_spec((3, 1)),
                                col3_spec],
        (col3_spec, col3_spec),
        (jax.ShapeDtypeStruct((3, T), f32), jax.ShapeDtypeStruct((3, T), f32)),
        z2p[0], z2p[1], y2, dinv, b2.reshape(1, HID), Wout,
        bout.reshape(3, 1), t3)

    t12o = tout3[0]
    t13o = tout3[1]
    t23o = tout3[2]

    d_all = jnp.concatenate([deltat.reshape(3 * T), jnp.zeros((cpad,), f32)])

    updp = _sc_scatter_upd(corr_all, d_all, zeros_m)

    SH = 6250
    ec_out = pl.pallas_call(
        _tc4_body,
        out_shape=jax.ShapeDtypeStruct((SH, WIN), f32),
    )(edge_costs.reshape(SH, WIN), edge_counter.reshape(SH, WIN),
      updp[0].reshape(SH, WIN), updp[1].reshape(SH, WIN))

    return (ec_out.reshape(M), t12o, t13o, t23o)


# speedup vs baseline: 36.2282x; 1.0636x over previous
import functools

import jax
import jax.numpy as jnp
from jax import lax
from jax.experimental import pallas as pl
from jax.experimental.pallas import tpu as pltpu
from jax.experimental.pallas import tpu_sc as plsc

T = 50000
M = 800000
E = 800000
HID = 32
NC = 2
NS = 16
NW = NC * NS
WIN = 128
TP = T + WIN

K = 4
SBE = K * WIN
WIN2 = 512
WINA = 256
NSET = 3
EPW = 25088
EP = NW * EPW
NSB = 49
NSTEP = EPW // WINA

CPW = 4736
CP = NW * CPW
NSBC = 9
TAILC = 1

_mesh = plsc.VectorSubcoreMesh(core_axis_name="c", subcore_axis_name="s",
                               num_cores=NC, num_subcores=NS)


@functools.partial(
    pl.kernel,
    out_type=(
        jax.ShapeDtypeStruct((CP,), jnp.float32),
        jax.ShapeDtypeStruct((NC, T), jnp.float32),
    ),
    mesh=_mesh,
    scratch_types=(
        pltpu.VMEM((K, WIN), jnp.int32),
        pltpu.VMEM((K, WIN), jnp.float32),
        pltpu.VMEM((K, WIN), jnp.float32),
        pltpu.VMEM((K, WIN), jnp.float32),
        pltpu.VMEM((K, WIN), jnp.int32),
        pltpu.VMEM((WIN,), jnp.float32),
        pltpu.VMEM_SHARED((TP,), jnp.float32),
        pltpu.SemaphoreType.DMA,
        pltpu.SemaphoreType.DMA,
    ),
    compiler_params=pltpu.CompilerParams(use_tc_tiling_on_sc=False),
)
def _sc_phase1(ec_hbm, cnt_hbm, corr_hbm, t_hbm, dst_hbm, ztp_hbm,
               tout_hbm, deg_hbm,
               corr_v, ecg_v, cntg_v, t_v, dst_v, ones_v, deg_sh, gsem, ssem):
    c = lax.axis_index("c")
    s = lax.axis_index("s")
    wid = s * NC + c

    @pl.when(s == 0)
    def _():
        pltpu.sync_copy(ztp_hbm, deg_sh)

    for i in range(WIN // 16):
        ones_v[pl.ds(i * 16, 16)] = jnp.full((16,), 1.0, jnp.float32)

    plsc.subcore_barrier()

    def corr_superbatch(base, nk):
        descs = []
        for kk in range(nk):
            descs.append(pltpu.async_copy(
                corr_hbm.at[pl.ds(base + kk * WIN, WIN)], corr_v.at[kk], gsem))
            descs.append(pltpu.async_copy(
                t_hbm.at[pl.ds(base + kk * WIN, WIN)], t_v.at[kk], gsem))
        for d in descs:
            d.wait()
        descs = []
        for kk in range(nk):
            descs.append(pltpu.async_copy(
                ec_hbm.at[corr_v.at[kk]], ecg_v.at[kk], gsem))
            descs.append(pltpu.async_copy(
                cnt_hbm.at[corr_v.at[kk]], cntg_v.at[kk], gsem))
        for d in descs:
            d.wait()
        for kk in range(nk):
            for i in range(WIN // 16):
                ds = pl.ds(i * 16, 16)
                t_v[kk, ds] = t_v[kk, ds] + ecg_v[kk, ds] / cntg_v[kk, ds]
        descs = []
        for kk in range(nk):
            descs.append(pltpu.async_copy(
                t_v.at[kk], tout_hbm.at[pl.ds(base + kk * WIN, WIN)], ssem))
        for d in descs:
            d.wait()

    cbase = wid * CPW

    def corr_body(j, _):
        corr_superbatch(cbase + j * K * WIN, K)
        return 0

    lax.fori_loop(0, NSBC, corr_body, 0)
    corr_superbatch(cbase + NSBC * K * WIN, TAILC)

    ebase = wid * EPW

    def deg_superbatch(base, nk):
        descs = []
        for kk in range(nk):
            descs.append(pltpu.async_copy(
                dst_hbm.at[pl.ds(base + kk * WIN, WIN)], dst_v.at[kk], gsem))
        for d in descs:
            d.wait()
        descs = []
        for kk in range(nk):
            descs.append(pltpu.async_copy(
                ones_v, deg_sh.at[dst_v.at[kk]], ssem, add=True))
        for d in descs:
            d.wait()

    def deg_body(j, _):
        deg_superbatch(ebase + j * SBE, K)
        return 0

    lax.fori_loop(0, NSB, deg_body, 0)

    plsc.subcore_barrier()

    @pl.when(s == 0)
    def _():
        pltpu.sync_copy(deg_sh.at[pl.ds(0, T)], deg_hbm.at[c])


@functools.partial(
    pl.kernel,
    out_type=jax.ShapeDtypeStruct((NC, T, HID), jnp.float32),
    mesh=_mesh,
    scratch_types=(
        pltpu.VMEM((NSET, WINA), jnp.int32),
        pltpu.VMEM((NSET, WINA), jnp.int32),
        pltpu.VMEM((NSET, WINA, HID), jnp.float32),
        pltpu.VMEM_SHARED((TP, HID), jnp.float32),
        pltpu.SemaphoreType.DMA,
        pltpu.SemaphoreType.DMA,
        pltpu.SemaphoreType.DMA,
        pltpu.SemaphoreType.DMA,
        pltpu.SemaphoreType.DMA,
        pltpu.SemaphoreType.DMA,
        pltpu.SemaphoreType.DMA,
        pltpu.SemaphoreType.DMA,
        pltpu.SemaphoreType.DMA,
        pltpu.SemaphoreType.DMA,
        pltpu.SemaphoreType.DMA,
        pltpu.SemaphoreType.DMA,
    ),
    compiler_params=pltpu.CompilerParams(use_tc_tiling_on_sc=False),
)
def _sc_agg(y_hbm, src_hbm, dst_hbm, zt_hbm, z_hbm,
            src_v, dst_v, rows_v, z_sh,
            gsem_0, gsem_1, gsem_2, ssem_0, ssem_1, ssem_2,
            isem_0, isem_1, isem_2, dsem_0, dsem_1, dsem_2):
    c = lax.axis_index("c")
    s = lax.axis_index("s")
    wid = s * NC + c
    ebase = wid * EPW
    gsem = (gsem_0, gsem_1, gsem_2)
    ssem = (ssem_0, ssem_1, ssem_2)
    isem = (isem_0, isem_1, isem_2)
    dsem = (dsem_0, dsem_1, dsem_2)

    @pl.when(s == 0)
    def _():
        pltpu.sync_copy(zt_hbm, z_sh)

    plsc.subcore_barrier()

    def fire_src(step, S):
        base = ebase + step * WINA
        pltpu.async_copy(src_hbm.at[pl.ds(base, WINA)], src_v.at[S], isem[S])

    def drain_src(step, S):
        base = ebase + step * WINA
        pltpu.make_async_copy(src_hbm.at[pl.ds(base, WINA)], src_v.at[S],
                              isem[S]).wait()

    def fire_dst(step, S):
        base = ebase + step * WINA
        pltpu.async_copy(dst_hbm.at[pl.ds(base, WINA)], dst_v.at[S], dsem[S])

    def drain_dst(step, S):
        base = ebase + step * WINA
        pltpu.make_async_copy(dst_hbm.at[pl.ds(base, WINA)], dst_v.at[S],
                              dsem[S]).wait()

    def fire_gathers(step, S):
        pltpu.async_copy(y_hbm.at[src_v.at[S]], rows_v.at[S], gsem[S])

    def drain_gathers(S):
        pltpu.make_async_copy(y_hbm.at[src_v.at[S]], rows_v.at[S],
                              gsem[S]).wait()

    def fire_scatters(S):
        pltpu.async_copy(rows_v.at[S], z_sh.at[dst_v.at[S]], ssem[S],
                         add=True)

    def drain_scatters(S):
        pltpu.make_async_copy(rows_v.at[S], z_sh.at[dst_v.at[S]],
                              ssem[S]).wait()

    fire_src(0, 0)
    fire_src(1, 1)
    fire_src(2, 2)
    fire_dst(0, 0)
    fire_dst(1, 1)
    fire_dst(2, 2)
    drain_src(0, 0)
    fire_gathers(0, 0)
    drain_src(1, 1)
    fire_gathers(1, 1)
    drain_gathers(0)
    drain_dst(0, 0)
    fire_scatters(0)
    fire_src(3, 0)
    drain_src(2, 2)
    fire_gathers(2, 2)
    drain_gathers(1)
    drain_dst(1, 1)
    fire_scatters(1)
    fire_src(4, 1)

    def body(j, _):
        x0 = 3 + 3 * j
        for dx in range(3):
            x = x0 + dx
            S = dx
            P = (dx + 2) % 3
            drain_scatters(S)
            fire_dst(x, S)
            drain_src(x, S)
            fire_gathers(x, S)
            drain_gathers(P)
            drain_dst(x - 1, P)
            fire_scatters(P)
            fire_src(x + 2, P)
        return 0

    lax.fori_loop(0, (NSTEP - 5) // 3, body, 0)

    drain_scatters(0)
    fire_dst(NSTEP - 2, 0)
    drain_src(NSTEP - 2, 0)
    fire_gathers(NSTEP - 2, 0)
    drain_gathers(2)
    drain_dst(NSTEP - 3, 2)
    fire_scatters(2)
    drain_scatters(1)
    fire_dst(NSTEP - 1, 1)
    drain_src(NSTEP - 1, 1)
    fire_gathers(NSTEP - 1, 1)
    drain_gathers(0)
    drain_dst(NSTEP - 2, 0)
    fire_scatters(0)
    drain_gathers(1)
    drain_dst(NSTEP - 1, 1)
    fire_scatters(1)
    drain_scatters(2)
    drain_scatters(0)
    drain_scatters(1)

    plsc.subcore_barrier()

    @pl.when(s == 0)
    def _():
        pltpu.sync_copy(z_sh.at[pl.ds(0, T)], z_hbm.at[c])


@functools.partial(
    pl.kernel,
    out_type=jax.ShapeDtypeStruct((NC, M), jnp.float32),
    mesh=_mesh,
    scratch_types=(
        pltpu.VMEM((K, WIN), jnp.int32),
        pltpu.VMEM((K, WIN), jnp.float32),
        pltpu.VMEM_SHARED((M,), jnp.float32),
        pltpu.SemaphoreType.DMA,
        pltpu.SemaphoreType.DMA,
    ),
    compiler_params=pltpu.CompilerParams(use_tc_tiling_on_sc=False),
)
def _sc_scatter_upd(corr_hbm, d_hbm, zm_hbm, upd_hbm,
                    corr_v, d_v, upd_sh, gsem, ssem):
    c = lax.axis_index("c")
    s = lax.axis_index("s")
    wid = s * NC + c
    cbase = wid * CPW

    @pl.when(s == 0)
    def _():
        pltpu.sync_copy(zm_hbm, upd_sh)

    plsc.subcore_barrier()

    def superbatch(base, nk):
        descs = []
        for kk in range(nk):
            descs.append(pltpu.async_copy(
                corr_hbm.at[pl.ds(base + kk * WIN, WIN)], corr_v.at[kk], gsem))
            descs.append(pltpu.async_copy(
                d_hbm.at[pl.ds(base + kk * WIN, WIN)], d_v.at[kk], gsem))
        for d in descs:
            d.wait()
        descs = []
        for kk in range(nk):
            descs.append(pltpu.async_copy(
                d_v.at[kk], upd_sh.at[corr_v.at[kk]], ssem, add=True))
        for d in descs:
            d.wait()

    def body(j, _):
        superbatch(cbase + j * K * WIN, K)
        return 0

    lax.fori_loop(0, NSBC, body, 0)
    superbatch(cbase + NSBC * K * WIN, TAILC)

    plsc.subcore_barrier()

    @pl.when(s == 0)
    def _():
        pltpu.sync_copy(upd_sh, upd_hbm.at[c])


def _tc1_body(t3_ref, d0_ref, d1_ref, w1_ref, y1_ref, dinv_ref):
    deg = d0_ref[...] + d1_ref[...] + 1.0
    dinv = lax.rsqrt(deg)
    dinv_ref[...] = dinv
    y = lax.dot_general(t3_ref[...], w1_ref[...],
                        (((0,), (0,)), ((), ())),
                        preferred_element_type=jnp.float32)
    y1_ref[...] = dinv * y


def _tc2_body(z0_ref, z1_ref, y_ref, dinv_ref, b_ref, w2_ref, y2_ref):
    dinv = dinv_ref[...]
    x = jnp.maximum(dinv * (z0_ref[...] + z1_ref[...] + y_ref[...])
                    + b_ref[...], 0.0)
    h = jnp.dot(x, w2_ref[...], preferred_element_type=jnp.float32)
    y2_ref[...] = dinv * h


def _tc3_body(z0_ref, z1_ref, y_ref, dinv_ref, b_ref, wout_ref, boutt_ref,
              t3_ref, deltat_ref, tout3_ref):
    dinv = dinv_ref[...]
    x = jnp.maximum(dinv * (z0_ref[...] + z1_ref[...] + y_ref[...])
                    + b_ref[...], 0.0)
    deltat = lax.dot_general(wout_ref[...], x,
                             (((0,), (1,)), ((), ())),
                             preferred_element_type=jnp.float32)
    deltat = deltat + boutt_ref[...]
    deltat_ref[...] = deltat
    tout3_ref[...] = t3_ref[...] - deltat


def _tc4_body(ec_ref, cnt_ref, u0_ref, u1_ref, out_ref):
    ec = ec_ref[...]
    out_ref[...] = jnp.where(cnt_ref[...] > 0, 0.0, ec) + u0_ref[...] + u1_ref[...]


BLK = 2048
NBLK = (T + BLK - 1) // BLK


def _row_spec(width):
    return pl.BlockSpec((BLK, width), lambda i: (i, 0))


def _full_spec(shape):
    return pl.BlockSpec(shape, lambda i: (0, 0))


def _tc_call(body, in_specs, out_specs, out_shapes, *args):
    return pl.pallas_call(
        body,
        grid=(NBLK,),
        in_specs=in_specs,
        out_specs=out_specs,
        out_shape=out_shapes,
    )(*args)


def kernel(edge_costs, t12, t13, t23, corr_12, corr_13, corr_23, edge_counter, edge_index, W1, b1, W2, b2, Wout, bout):
    f32 = jnp.float32
    idt = edge_index.dtype
    npad = EP - E
    src = edge_index[0]
    dst = edge_index[1]
    src_p = jnp.concatenate([src, jnp.arange(npad, dtype=idt) % T])
    dst_p = jnp.concatenate([dst, T + jnp.arange(npad, dtype=idt) % WIN])

    cpad = CP - 3 * T
    corr_all = jnp.concatenate([
        corr_12, corr_13, corr_23,
        jnp.arange(cpad, dtype=corr_12.dtype) % M,
    ])
    t_all = jnp.concatenate([t12, t13, t23, jnp.zeros((cpad,), f32)])
    zeros_tp = jnp.zeros((TP,), f32)
    zeros_tp32 = jnp.zeros((TP, HID), f32)
    zeros_m = jnp.zeros((M,), f32)

    t_all_new, deg_part = _sc_phase1(
        edge_costs, edge_counter, corr_all, t_all, dst_p, zeros_tp)

    t3 = t_all_new[:3 * T].reshape(3, T)
    d0 = deg_part[0].reshape(T, 1)
    d1 = deg_part[1].reshape(T, 1)

    col3_spec = pl.BlockSpec((3, BLK), lambda i: (0, i))
    y1, dinv = _tc_call(
        _tc1_body,
        [col3_spec, _row_spec(1), _row_spec(1), _full_spec((3, HID))],
        (_row_spec(HID), _row_spec(1)),
        (jax.ShapeDtypeStruct((T, HID), f32), jax.ShapeDtypeStruct((T, 1), f32)),
        t3, d0, d1, W1)

    z1p = _sc_agg(y1, src_p, dst_p, zeros_tp32)

    y2 = _tc_call(
        _tc2_body,
        [_row_spec(HID)] * 3 + [_row_spec(1), _full_spec((1, HID)),
                                _full_spec((HID, HID))],
        _row_spec(HID),
        jax.ShapeDtypeStruct((T, HID), f32),
        z1p[0], z1p[1], y1, dinv, b1.reshape(1, HID), W2)

    z2p = _sc_agg(y2, src_p, dst_p, zeros_tp32)

    deltat, tout3 = _tc_call(
        _tc3_body,
        [_row_spec(HID)] * 3 + [_row_spec(1), _full_spec((1, HID)),
                                _full_spec((HID, 3)), _full_spec((3, 1)),
                                col3_spec],
        (col3_spec, col3_spec),
        (jax.ShapeDtypeStruct((3, T), f32), jax.ShapeDtypeStruct((3, T), f32)),
        z2p[0], z2p[1], y2, dinv, b2.reshape(1, HID), Wout,
        bout.reshape(3, 1), t3)

    t12o = tout3[0]
    t13o = tout3[1]
    t23o = tout3[2]

    d_all = jnp.concatenate([deltat.reshape(3 * T), jnp.zeros((cpad,), f32)])

    updp = _sc_scatter_upd(corr_all, d_all, zeros_m)

    SH = 6250
    ec_out = pl.pallas_call(
        _tc4_body,
        out_shape=jax.ShapeDtypeStruct((SH, WIN), f32),
    )(edge_costs.reshape(SH, WIN), edge_counter.reshape(SH, WIN),
      updp[0].reshape(SH, WIN), updp[1].reshape(SH, WIN))

    return (ec_out.reshape(M), t12o, t13o, t23o)
